# baseline (speedup 1.0000x reference)
import functools

import jax
import jax.numpy as jnp
from jax import lax
from jax.experimental import pallas as pl
from jax.experimental.pallas import tpu as pltpu
from jax.experimental.pallas import tpu_sc as plsc

_N = 10000
_E = 640000
_D_IN = 128
_DH = 64
_NCLS = 2

_NC = 2
_NS = 16
_CH = 125
_CPT = _E // (_NC * _NS * _CH)
_NP = 10240
_RPT = _NP // _NS

_mesh = plsc.VectorSubcoreMesh(core_axis_name="c", subcore_axis_name="s")


_NBUF = 4


def _make_agg(with_deg):
    out_type = [jax.ShapeDtypeStruct((_NC, _NP, _DH), jnp.float32)]
    scratch = [
        pltpu.VMEM((_CPT, _CH), jnp.int32),
        pltpu.VMEM((_CPT, _CH), jnp.int32),
    ] + [pltpu.VMEM((_CH, _DH), jnp.float32)] * _NBUF + [
        pltpu.VMEM_SHARED((_NP, _DH), jnp.float32),
    ] + [pltpu.SemaphoreType.DMA] * (2 * _NBUF)
    if with_deg:
        out_type.append(jax.ShapeDtypeStruct((_NC, _NP, 16), jnp.float32))
        scratch += [
            pltpu.VMEM((_CH, 16), jnp.float32),
            pltpu.VMEM_SHARED((_NP, 16), jnp.float32),
        ]

    def body(z_hbm, src_hbm, dst_hbm, zeros_hbm, *rest):
        if with_deg:
            (ones_hbm, zeros16_hbm, out_hbm, deg_out_hbm,
             idx_s, idx_d, *bufs) = rest
            rows = bufs[:_NBUF]
            agg_sh = bufs[_NBUF]
            sem_g = bufs[_NBUF + 1:_NBUF + 1 + _NBUF]
            sem_s = bufs[_NBUF + 1 + _NBUF:_NBUF + 1 + 2 * _NBUF]
            ones_v, deg_sh = bufs[-2:]
        else:
            out_hbm, idx_s, idx_d, *bufs = rest
            rows = bufs[:_NBUF]
            agg_sh = bufs[_NBUF]
            sem_g = bufs[_NBUF + 1:_NBUF + 1 + _NBUF]
            sem_s = bufs[_NBUF + 1 + _NBUF:]
        cid = lax.axis_index("c")
        sid = lax.axis_index("s")
        r0 = sid * _RPT
        pltpu.sync_copy(zeros_hbm.at[pl.ds(r0, _RPT)],
                        agg_sh.at[pl.ds(r0, _RPT)])
        base = (cid * _NS + sid) * _CPT
        pltpu.sync_copy(src_hbm.at[pl.ds(base, _CPT)], idx_s)
        pltpu.sync_copy(dst_hbm.at[pl.ds(base, _CPT)], idx_d)
        if with_deg:
            pltpu.sync_copy(zeros16_hbm.at[pl.ds(r0, _RPT)],
                            deg_sh.at[pl.ds(r0, _RPT)])
            pltpu.sync_copy(ones_hbm, ones_v)
        plsc.subcore_barrier()

        def gather(i, b):
            pltpu.async_copy(z_hbm.at[idx_s.at[i]], rows[b], sem_g[b])

        def wait_gather(i, b):
            pltpu.make_async_copy(z_hbm.at[idx_s.at[i]], rows[b],
                                  sem_g[b]).wait()

        def scatter(i, b):
            pltpu.async_copy(rows[b], agg_sh.at[idx_d.at[i]], sem_s[b],
                             add=True)

        def wait_scatter(i, b):
            pltpu.make_async_copy(rows[b], agg_sh.at[idx_d.at[i]],
                                  sem_s[b]).wait()

        gather(0, 0)
        gather(1, 1)

        def loop_body(j, carry):
            i0 = _NBUF * j
            for b in range(_NBUF):
                i = i0 + b

                @pl.when(i >= 2)
                def _():
                    wait_scatter(i - 2, (b - 2) % _NBUF)

                @pl.when(i + 2 < _CPT)
                def _():
                    gather(i + 2, (b + 2) % _NBUF)

                wait_gather(i, b)
                if with_deg:
                    pltpu.sync_copy(ones_v, deg_sh.at[idx_d.at[i]],
                                    add=True)
                scatter(i, b)
            return carry

        lax.fori_loop(0, _CPT // _NBUF, loop_body, 0)
        wait_scatter(_CPT - 2, (_CPT - 2) % _NBUF)
        wait_scatter(_CPT - 1, (_CPT - 1) % _NBUF)
        plsc.subcore_barrier()
        pltpu.sync_copy(agg_sh.at[pl.ds(r0, _RPT)],
                        out_hbm.at[cid, pl.ds(r0, _RPT)])
        if with_deg:
            pltpu.sync_copy(deg_sh.at[pl.ds(r0, _RPT)],
                            deg_out_hbm.at[cid, pl.ds(r0, _RPT)])

    return pl.kernel(
        body,
        out_type=tuple(out_type),
        mesh=_mesh,
        compiler_params=pltpu.CompilerParams(use_tc_tiling_on_sc=False),
        scratch_types=scratch,
    )


_agg_deg_kernel = _make_agg(True)
_agg_kernel = _make_agg(False)


def _mm_body(x_ref, w_ref, o_ref):
    o_ref[...] = jnp.dot(x_ref[...], w_ref[...],
                         preferred_element_type=jnp.float32)


_mm = pl.pallas_call(
    _mm_body,
    out_shape=jax.ShapeDtypeStruct((_N, _DH), jnp.float32),
)


def _fuse_z_body(s_ref, p_ref, deg_ref, b_ref, wn_ref, z_ref, h_ref):
    deg = deg_ref[0, 0:_N, 0:1] + deg_ref[1, 0:_N, 0:1]
    inv = 1.0 / jnp.maximum(deg, 1.0)
    h = s_ref[...] + (p_ref[0, 0:_N] + p_ref[1, 0:_N]) * inv + b_ref[...]
    h = jnp.maximum(h, 0.0)
    h_ref[...] = h
    z_ref[...] = jnp.dot(h, wn_ref[...], preferred_element_type=jnp.float32)


_fuse_z = pl.pallas_call(
    _fuse_z_body,
    out_shape=(
        jax.ShapeDtypeStruct((_N, _DH), jnp.float32),
        jax.ShapeDtypeStruct((_N, _DH), jnp.float32),
    ),
)


def _final_body(s_ref, p_ref, deg_ref, b_ref, wc_ref, bc_ref,
                out_ref, feat_ref, h_ref):
    deg = deg_ref[0, 0:_N, 0:1] + deg_ref[1, 0:_N, 0:1]
    inv = 1.0 / jnp.maximum(deg, 1.0)
    h = s_ref[...] + (p_ref[0, 0:_N] + p_ref[1, 0:_N]) * inv + b_ref[...]
    h_ref[...] = h
    feat = jnp.sum(h, axis=0, keepdims=True) * (1.0 / _N)
    feat_ref[...] = feat
    out_ref[...] = jnp.dot(feat, wc_ref[...],
                           preferred_element_type=jnp.float32) + bc_ref[...]


_final = pl.pallas_call(
    _final_body,
    out_shape=(
        jax.ShapeDtypeStruct((1, _NCLS), jnp.float32),
        jax.ShapeDtypeStruct((1, _DH), jnp.float32),
        jax.ShapeDtypeStruct((_N, _DH), jnp.float32),
    ),
)


def kernel(x, edge_index, W_self1, W_neigh1, b1, W_self2, W_neigh2, b2,
           W_self3, W_neigh3, b3, W_cls, b_cls):
    ei = edge_index.astype(jnp.int32)
    src2d = ei[0].reshape(_E // _CH, _CH)
    dst2d = ei[1].reshape(_E // _CH, _CH)
    zeros64 = jnp.zeros((_NP, _DH), jnp.float32)
    zeros16 = jnp.zeros((_NP, 16), jnp.float32)
    ones16 = jnp.ones((_CH, 16), jnp.float32)

    z1 = _mm(x, W_neigh1)
    p1, deg16 = _agg_deg_kernel(z1, src2d, dst2d, zeros64, ones16, zeros16)
    s1 = _mm(x, W_self1)
    z2, h2 = _fuse_z(s1, p1, deg16, b1.reshape(1, _DH), W_neigh2)
    p2, = _agg_kernel(z2, src2d, dst2d, zeros64)
    s2 = _mm(h2, W_self2)
    z3, h3 = _fuse_z(s2, p2, deg16, b2.reshape(1, _DH), W_neigh3)
    p3, = _agg_kernel(z3, src2d, dst2d, zeros64)
    s3 = _mm(h3, W_self3)
    out, feat, h = _final(s3, p3, deg16, b3.reshape(1, _DH),
                          W_cls, b_cls.reshape(1, _NCLS))
    return (out, feat, h)


# speedup vs baseline: 1.0847x; 1.0847x over previous
import functools

import numpy as np

import jax
import jax.numpy as jnp
from jax import lax
from jax.experimental import pallas as pl
from jax.experimental.pallas import tpu as pltpu
from jax.experimental.pallas import tpu_sc as plsc

_N = 10000
_E = 640000
_D_IN = 128
_DH = 64
_NCLS = 2

_NC = 2
_NS = 16
_CH = 128
_EP = 655360
_CPT = _EP // (_NC * _NS * _CH)
_NP = 10240
_RPT = _NP // _NS
_NBUF = 4

_mesh = plsc.VectorSubcoreMesh(core_axis_name="c", subcore_axis_name="s")


def _make_agg(with_deg):
    out_type = [jax.ShapeDtypeStruct((_NC, _NP, _DH), jnp.float32)]
    scratch = [
        pltpu.VMEM((_CPT, _CH), jnp.int32),
        pltpu.VMEM((_CPT, _CH), jnp.int32),
    ] + [pltpu.VMEM((_CH, _DH), jnp.float32)] * _NBUF + [
        pltpu.VMEM_SHARED((_NP, _DH), jnp.float32),
    ] + [pltpu.SemaphoreType.DMA] * (2 * _NBUF)
    if with_deg:
        out_type.append(jax.ShapeDtypeStruct((_NC, _NP, 16), jnp.float32))
        scratch += [
            pltpu.VMEM((_CH, 16), jnp.float32),
            pltpu.VMEM_SHARED((_NP, 16), jnp.float32),
        ]

    def body(z_hbm, src_hbm, dst_hbm, zeros_hbm, *rest):
        if with_deg:
            (ones_hbm, zeros16_hbm, out_hbm, deg_out_hbm,
             idx_s, idx_d, *bufs) = rest
            rows = bufs[:_NBUF]
            agg_sh = bufs[_NBUF]
            sem_g = bufs[_NBUF + 1:_NBUF + 1 + _NBUF]
            sem_s = bufs[_NBUF + 1 + _NBUF:_NBUF + 1 + 2 * _NBUF]
            ones_v, deg_sh = bufs[-2:]
        else:
            out_hbm, idx_s, idx_d, *bufs = rest
            rows = bufs[:_NBUF]
            agg_sh = bufs[_NBUF]
            sem_g = bufs[_NBUF + 1:_NBUF + 1 + _NBUF]
            sem_s = bufs[_NBUF + 1 + _NBUF:]
        cid = lax.axis_index("c")
        sid = lax.axis_index("s")
        r0 = sid * _RPT
        pltpu.sync_copy(zeros_hbm.at[pl.ds(r0, _RPT)],
                        agg_sh.at[pl.ds(r0, _RPT)])
        base = (cid * _NS + sid) * _CPT
        pltpu.sync_copy(src_hbm.at[pl.ds(base, _CPT)], idx_s)
        pltpu.sync_copy(dst_hbm.at[pl.ds(base, _CPT)], idx_d)
        if with_deg:
            pltpu.sync_copy(zeros16_hbm.at[pl.ds(r0, _RPT)],
                            deg_sh.at[pl.ds(r0, _RPT)])
            pltpu.sync_copy(ones_hbm, ones_v)
        plsc.subcore_barrier()

        def gather(i, b):
            pltpu.async_copy(z_hbm.at[idx_s.at[i]], rows[b], sem_g[b])

        def wait_gather(i, b):
            pltpu.make_async_copy(z_hbm.at[idx_s.at[i]], rows[b],
                                  sem_g[b]).wait()

        def scatter(i, b):
            pltpu.async_copy(rows[b], agg_sh.at[idx_d.at[i]], sem_s[b],
                             add=True)

        def wait_scatter(i, b):
            pltpu.make_async_copy(rows[b], agg_sh.at[idx_d.at[i]],
                                  sem_s[b]).wait()

        gather(0, 0)
        gather(1, 1)

        def loop_body(j, carry):
            i0 = _NBUF * j
            for b in range(_NBUF):
                i = i0 + b

                @pl.when(i >= 2)
                def _():
                    wait_scatter(i - 2, (b - 2) % _NBUF)

                @pl.when(i + 2 < _CPT)
                def _():
                    gather(i + 2, (b + 2) % _NBUF)

                wait_gather(i, b)
                if with_deg:
                    pltpu.sync_copy(ones_v, deg_sh.at[idx_d.at[i]],
                                    add=True)
                scatter(i, b)
            return carry

        lax.fori_loop(0, _CPT // _NBUF, loop_body, 0)
        wait_scatter(_CPT - 2, (_CPT - 2) % _NBUF)
        wait_scatter(_CPT - 1, (_CPT - 1) % _NBUF)
        plsc.subcore_barrier()
        pltpu.sync_copy(agg_sh.at[pl.ds(r0, _RPT)],
                        out_hbm.at[cid, pl.ds(r0, _RPT)])
        if with_deg:
            pltpu.sync_copy(deg_sh.at[pl.ds(r0, _RPT)],
                            deg_out_hbm.at[cid, pl.ds(r0, _RPT)])

    return pl.kernel(
        body,
        out_type=tuple(out_type),
        mesh=_mesh,
        compiler_params=pltpu.CompilerParams(use_tc_tiling_on_sc=False),
        scratch_types=scratch,
    )


_agg_deg_kernel = _make_agg(True)
_agg_kernel = _make_agg(False)


def _proj_body(x_ref, wn_ref, ws_ref, zs_ref):
    x = x_ref[...]
    zs_ref[:, 0:_DH] = jnp.dot(x, wn_ref[...],
                               preferred_element_type=jnp.float32)
    zs_ref[:, _DH:2 * _DH] = jnp.dot(x, ws_ref[...],
                                     preferred_element_type=jnp.float32)


_proj = pl.pallas_call(
    _proj_body,
    out_shape=jax.ShapeDtypeStruct((_N, 2 * _DH), jnp.float32),
)


def _fuse_body(zs_ref, p_ref, deg_ref, b_ref, wn_ref, ws_ref, zs2_ref):
    deg = deg_ref[0, 0:_N, 0:1] + deg_ref[1, 0:_N, 0:1]
    inv = 1.0 / jnp.maximum(deg, 1.0)
    s = zs_ref[:, _DH:2 * _DH]
    h = s + (p_ref[0, 0:_N] + p_ref[1, 0:_N]) * inv + b_ref[...]
    h = jnp.maximum(h, 0.0)
    zs2_ref[:, 0:_DH] = jnp.dot(h, wn_ref[...],
                                preferred_element_type=jnp.float32)
    zs2_ref[:, _DH:2 * _DH] = jnp.dot(h, ws_ref[...],
                                      preferred_element_type=jnp.float32)


_fuse = pl.pallas_call(
    _fuse_body,
    out_shape=jax.ShapeDtypeStruct((_N, 2 * _DH), jnp.float32),
)


def _final_body(zs_ref, p_ref, deg_ref, b_ref, wc_ref, bc_ref,
                out_ref, feat_ref, h_ref):
    deg = deg_ref[0, 0:_N, 0:1] + deg_ref[1, 0:_N, 0:1]
    inv = 1.0 / jnp.maximum(deg, 1.0)
    s = zs_ref[:, _DH:2 * _DH]
    h = s + (p_ref[0, 0:_N] + p_ref[1, 0:_N]) * inv + b_ref[...]
    h_ref[...] = h
    feat = jnp.sum(h, axis=0, keepdims=True) * (1.0 / _N)
    feat_ref[...] = feat
    out_ref[...] = jnp.dot(feat, wc_ref[...],
                           preferred_element_type=jnp.float32) + bc_ref[...]


_final = pl.pallas_call(
    _final_body,
    out_shape=(
        jax.ShapeDtypeStruct((1, _NCLS), jnp.float32),
        jax.ShapeDtypeStruct((1, _DH), jnp.float32),
        jax.ShapeDtypeStruct((_N, _DH), jnp.float32),
    ),
)

_PAD_SRC = np.arange(_EP - _E, dtype=np.int32) % _N
_PAD_DST = _N + np.arange(_EP - _E, dtype=np.int32) % (_NP - _N)


def kernel(x, edge_index, W_self1, W_neigh1, b1, W_self2, W_neigh2, b2,
           W_self3, W_neigh3, b3, W_cls, b_cls):
    ei = edge_index.astype(jnp.int32)
    src2d = (jnp.concatenate([ei[0], jnp.asarray(_PAD_SRC)]) * 2
             ).reshape(_EP // _CH, _CH)
    dst2d = jnp.concatenate([ei[1], jnp.asarray(_PAD_DST)]
                            ).reshape(_EP // _CH, _CH)
    zeros64 = jnp.zeros((_NP, _DH), jnp.float32)
    zeros16 = jnp.zeros((_NP, 16), jnp.float32)
    ones16 = jnp.ones((_CH, 16), jnp.float32)

    zs1 = _proj(x, W_neigh1, W_self1)
    p1, deg16 = _agg_deg_kernel(zs1.reshape(2 * _N, _DH), src2d, dst2d,
                                zeros64, ones16, zeros16)
    zs2 = _fuse(zs1, p1, deg16, b1.reshape(1, _DH), W_neigh2, W_self2)
    p2, = _agg_kernel(zs2.reshape(2 * _N, _DH), src2d, dst2d, zeros64)
    zs3 = _fuse(zs2, p2, deg16, b2.reshape(1, _DH), W_neigh3, W_self3)
    p3, = _agg_kernel(zs3.reshape(2 * _N, _DH), src2d, dst2d, zeros64)
    out, feat, h = _final(zs3, p3, deg16, b3.reshape(1, _DH),
                          W_cls, b_cls.reshape(1, _NCLS))
    return (out, feat, h)


# speedup vs baseline: 1.1010x; 1.0150x over previous
import functools

import numpy as np

import jax
import jax.numpy as jnp
from jax import lax
from jax.experimental import pallas as pl
from jax.experimental.pallas import tpu as pltpu
from jax.experimental.pallas import tpu_sc as plsc

_N = 10000
_E = 640000
_D_IN = 128
_DH = 64
_NCLS = 2

_NC = 2
_NS = 16
_CH = 128
_NCHUNK = _E // _CH
_CPT = 156
_NP = 10240
_RPT = _NP // _NS
_NBUF = 4

_mesh = plsc.VectorSubcoreMesh(core_axis_name="c", subcore_axis_name="s")


def _make_agg(with_deg):
    out_type = [jax.ShapeDtypeStruct((_NC, _NP, _DH), jnp.float32)]
    scratch = [
        pltpu.VMEM((_CPT + 1, _CH), jnp.int32),
        pltpu.VMEM((_CPT + 1, _CH), jnp.int32),
    ] + [pltpu.VMEM((_CH, _DH), jnp.float32)] * _NBUF + [
        pltpu.VMEM_SHARED((_NP, _DH), jnp.float32),
    ] + [pltpu.SemaphoreType.DMA] * (2 * _NBUF)
    if with_deg:
        out_type.append(jax.ShapeDtypeStruct((_NC, _NP, 16), jnp.float32))
        scratch += [
            pltpu.VMEM((_CH, 16), jnp.float32),
            pltpu.VMEM_SHARED((_NP, 16), jnp.float32),
        ]

    def body(z_hbm, src_hbm, dst_hbm, zeros_hbm, *rest):
        if with_deg:
            (ones_hbm, zeros16_hbm, out_hbm, deg_out_hbm,
             idx_s, idx_d, *bufs) = rest
            rows = bufs[:_NBUF]
            agg_sh = bufs[_NBUF]
            sem_g = bufs[_NBUF + 1:_NBUF + 1 + _NBUF]
            sem_s = bufs[_NBUF + 1 + _NBUF:_NBUF + 1 + 2 * _NBUF]
            ones_v, deg_sh = bufs[-2:]
        else:
            out_hbm, idx_s, idx_d, *bufs = rest
            rows = bufs[:_NBUF]
            agg_sh = bufs[_NBUF]
            sem_g = bufs[_NBUF + 1:_NBUF + 1 + _NBUF]
            sem_s = bufs[_NBUF + 1 + _NBUF:]
        cid = lax.axis_index("c")
        sid = lax.axis_index("s")
        r0 = sid * _RPT
        pltpu.sync_copy(zeros_hbm.at[pl.ds(r0, _RPT)],
                        agg_sh.at[pl.ds(r0, _RPT)])
        tid = cid * _NS + sid
        base = _CPT * tid + jnp.maximum(tid - 24, 0)
        pltpu.sync_copy(src_hbm.at[pl.ds(base, _CPT + 1)], idx_s)
        pltpu.sync_copy(dst_hbm.at[pl.ds(base, _CPT + 1)], idx_d)
        if with_deg:
            pltpu.sync_copy(zeros16_hbm.at[pl.ds(r0, _RPT)],
                            deg_sh.at[pl.ds(r0, _RPT)])
            pltpu.sync_copy(ones_hbm, ones_v)
        plsc.subcore_barrier()

        def gather(i, b):
            pltpu.async_copy(z_hbm.at[idx_s.at[i]], rows[b], sem_g[b])

        def wait_gather(i, b):
            pltpu.make_async_copy(z_hbm.at[idx_s.at[i]], rows[b],
                                  sem_g[b]).wait()

        def scatter(i, b):
            pltpu.async_copy(rows[b], agg_sh.at[idx_d.at[i]], sem_s[b],
                             add=True)

        def wait_scatter(i, b):
            pltpu.make_async_copy(rows[b], agg_sh.at[idx_d.at[i]],
                                  sem_s[b]).wait()

        gather(0, 0)
        gather(1, 1)

        def loop_body(j, carry):
            i0 = _NBUF * j
            for b in range(_NBUF):
                i = i0 + b

                @pl.when(i >= 2)
                def _():
                    wait_scatter(i - 2, (b - 2) % _NBUF)

                @pl.when(i + 2 < _CPT)
                def _():
                    gather(i + 2, (b + 2) % _NBUF)

                wait_gather(i, b)
                if with_deg:
                    pltpu.sync_copy(ones_v, deg_sh.at[idx_d.at[i]],
                                    add=True)
                scatter(i, b)
            return carry

        lax.fori_loop(0, _CPT // _NBUF, loop_body, 0)
        wait_scatter(_CPT - 2, (_CPT - 2) % _NBUF)
        wait_scatter(_CPT - 1, (_CPT - 1) % _NBUF)

        @pl.when(tid >= 24)
        def _():
            gather(_CPT, 0)
            wait_gather(_CPT, 0)
            if with_deg:
                pltpu.sync_copy(ones_v, deg_sh.at[idx_d.at[_CPT]],
                                add=True)
            scatter(_CPT, 0)
            wait_scatter(_CPT, 0)

        plsc.subcore_barrier()
        pltpu.sync_copy(agg_sh.at[pl.ds(r0, _RPT)],
                        out_hbm.at[cid, pl.ds(r0, _RPT)])
        if with_deg:
            pltpu.sync_copy(deg_sh.at[pl.ds(r0, _RPT)],
                            deg_out_hbm.at[cid, pl.ds(r0, _RPT)])

    return pl.kernel(
        body,
        out_type=tuple(out_type),
        mesh=_mesh,
        compiler_params=pltpu.CompilerParams(use_tc_tiling_on_sc=False),
        scratch_types=scratch,
    )


_agg_deg_kernel = _make_agg(True)
_agg_kernel = _make_agg(False)


def _proj_body(x_ref, wn_ref, ws_ref, zs_ref):
    x = x_ref[...]
    zs_ref[:, 0:_DH] = jnp.dot(x, wn_ref[...],
                               preferred_element_type=jnp.float32)
    zs_ref[:, _DH:2 * _DH] = jnp.dot(x, ws_ref[...],
                                     preferred_element_type=jnp.float32)


_proj = pl.pallas_call(
    _proj_body,
    out_shape=jax.ShapeDtypeStruct((_N, 2 * _DH), jnp.float32),
)


def _fuse_body(zs_ref, p_ref, deg_ref, b_ref, wn_ref, ws_ref, zs2_ref):
    deg = deg_ref[0, 0:_N, 0:1] + deg_ref[1, 0:_N, 0:1]
    inv = 1.0 / jnp.maximum(deg, 1.0)
    s = zs_ref[:, _DH:2 * _DH]
    h = s + (p_ref[0, 0:_N] + p_ref[1, 0:_N]) * inv + b_ref[...]
    h = jnp.maximum(h, 0.0)
    zs2_ref[:, 0:_DH] = jnp.dot(h, wn_ref[...],
                                preferred_element_type=jnp.float32)
    zs2_ref[:, _DH:2 * _DH] = jnp.dot(h, ws_ref[...],
                                      preferred_element_type=jnp.float32)


_fuse = pl.pallas_call(
    _fuse_body,
    out_shape=jax.ShapeDtypeStruct((_N, 2 * _DH), jnp.float32),
)


def _final_body(zs_ref, p_ref, deg_ref, b_ref, wc_ref, bc_ref,
                out_ref, feat_ref, h_ref):
    deg = deg_ref[0, 0:_N, 0:1] + deg_ref[1, 0:_N, 0:1]
    inv = 1.0 / jnp.maximum(deg, 1.0)
    s = zs_ref[:, _DH:2 * _DH]
    h = s + (p_ref[0, 0:_N] + p_ref[1, 0:_N]) * inv + b_ref[...]
    h_ref[...] = h
    feat = jnp.sum(h, axis=0, keepdims=True) * (1.0 / _N)
    feat_ref[...] = feat
    out_ref[...] = jnp.dot(feat, wc_ref[...],
                           preferred_element_type=jnp.float32) + bc_ref[...]


_final = pl.pallas_call(
    _final_body,
    out_shape=(
        jax.ShapeDtypeStruct((1, _NCLS), jnp.float32),
        jax.ShapeDtypeStruct((1, _DH), jnp.float32),
        jax.ShapeDtypeStruct((_N, _DH), jnp.float32),
    ),
)

def kernel(x, edge_index, W_self1, W_neigh1, b1, W_self2, W_neigh2, b2,
           W_self3, W_neigh3, b3, W_cls, b_cls):
    ei = edge_index.astype(jnp.int32)
    src2d = (ei[0] * 2).reshape(_NCHUNK, _CH)
    dst2d = ei[1].reshape(_NCHUNK, _CH)
    zeros64 = jnp.zeros((_NP, _DH), jnp.float32)
    zeros16 = jnp.zeros((_NP, 16), jnp.float32)
    ones16 = jnp.ones((_CH, 16), jnp.float32)

    zs1 = _proj(x, W_neigh1, W_self1)
    p1, deg16 = _agg_deg_kernel(zs1.reshape(2 * _N, _DH), src2d, dst2d,
                                zeros64, ones16, zeros16)
    zs2 = _fuse(zs1, p1, deg16, b1.reshape(1, _DH), W_neigh2, W_self2)
    p2, = _agg_kernel(zs2.reshape(2 * _N, _DH), src2d, dst2d, zeros64)
    zs3 = _fuse(zs2, p2, deg16, b2.reshape(1, _DH), W_neigh3, W_self3)
    p3, = _agg_kernel(zs3.reshape(2 * _N, _DH), src2d, dst2d, zeros64)
    out, feat, h = _final(zs3, p3, deg16, b3.reshape(1, _DH),
                          W_cls, b_cls.reshape(1, _NCLS))
    return (out, feat, h)
